# baseline (speedup 1.0000x reference)
import functools

import jax
import jax.numpy as jnp
from jax.experimental import pallas as pl
from jax.experimental.pallas import tpu as pltpu

NUM_AGENTS = 8
HIDDEN = 1024
NZ = 2048
BATCH = 16384
ROWS = 1024
NZT = BATCH // ROWS


def _fused_kernel(aidx_ref, b_ref, w_ref, z_ref, out_ref, masks_ref, scale_ref):
    step = pl.program_id(0)

    @pl.when(step < NUM_AGENTS)
    def _reduce():
        m = (jnp.max(b_ref[0], axis=0) > 0).astype(jnp.float32)
        masks_ref[pl.ds(step, 1), :] = m[None, :]

        @pl.when(step == NUM_AGENTS - 1)
        def _finalize():
            alpha = jnp.sum(masks_ref[...], axis=0)
            mask_sel = masks_ref[pl.ds(aidx_ref[0], 1), :][0]
            weights = jnp.zeros((NZ,), jnp.float32)
            for k in range(NUM_AGENTS + 1):
                weights = jnp.where(alpha == float(k), w_ref[0, k], weights)
            scale_ref[0, :] = mask_sel * weights

    @pl.when(step >= NUM_AGENTS)
    def _mul():
        out_ref[...] = z_ref[...] * scale_ref[...]


@functools.partial(jax.jit, static_argnames=())
def kernel(Z_hat, B, w, agent_idx):
    B3 = B.reshape(NUM_AGENTS, HIDDEN, NZ)
    w2 = jnp.zeros((1, 16), jnp.float32).at[0, : NUM_AGENTS + 1].set(w)
    aidx = jnp.asarray(agent_idx, jnp.int32).reshape((1,))

    out = pl.pallas_call(
        _fused_kernel,
        grid_spec=pltpu.PrefetchScalarGridSpec(
            num_scalar_prefetch=1,
            grid=(NUM_AGENTS + NZT,),
            in_specs=[
                pl.BlockSpec(
                    (1, HIDDEN, NZ),
                    lambda s, aidx: (jnp.minimum(s, NUM_AGENTS - 1), 0, 0),
                ),
                pl.BlockSpec((1, 16), lambda s, aidx: (0, 0)),
                pl.BlockSpec(
                    (ROWS, NZ),
                    lambda s, aidx: (jnp.maximum(s - NUM_AGENTS, 0), 0),
                ),
            ],
            out_specs=pl.BlockSpec(
                (ROWS, NZ),
                lambda s, aidx: (jnp.maximum(s - NUM_AGENTS, 0), 0),
            ),
            scratch_shapes=[
                pltpu.VMEM((NUM_AGENTS, NZ), jnp.float32),
                pltpu.VMEM((1, NZ), jnp.float32),
            ],
        ),
        out_shape=jax.ShapeDtypeStruct((BATCH, NZ), jnp.float32),
    )(aidx, B3, w2, Z_hat)
    return out
